# Initial kernel scaffold; baseline (speedup 1.0000x reference)
import jax
import jax.numpy as jnp
from jax.experimental import pallas as pl


def kernel(t, y, x, edge_index, edge_attr, Wn, bn, We, be, Wm, Wu, Wo, bo, Wp1, bp1, Wp2, bp2):
    raise NotImplementedError("write your pallas kernel here")


# speedup vs baseline: 2.1741x; 2.1741x over previous
import functools

import jax
import jax.numpy as jnp
from jax import lax
from jax.experimental import pallas as pl
from jax.experimental.pallas import tpu as pltpu
from jax.experimental.pallas import tpu_sc as plsc

NN = 10000
EE = 320000
H = 128
LL = 4

NC = 2
NS = 16
NW = NC * NS
PER_W = EE // NW
K = 80
ITERS = PER_W // K
WB = 80
NCH = NN // WB
CPS = -(-NCH // NS)
BE = 4000
GE = EE // BE


def _mesh():
    return plsc.VectorSubcoreMesh(core_axis_name="c", subcore_axis_name="s")


@functools.partial(jax.jit, static_argnames=("w",))
def _sc_gather(table, idx, w):
    @functools.partial(
        pl.kernel,
        out_type=jax.ShapeDtypeStruct((EE, w), jnp.float32),
        mesh=_mesh(),
        scratch_types=[
            pltpu.VMEM((K,), jnp.int32),
            pltpu.VMEM((K, w), jnp.float32),
            pltpu.SemaphoreType.DMA,
        ],
    )
    def k(table_hbm, idx_hbm, out_hbm, idx_v, rows_v, sem):
        cid = lax.axis_index("c")
        sid = lax.axis_index("s")
        base = (sid * NC + cid) * PER_W

        def body(i, _):
            off = base + i * K
            pltpu.sync_copy(idx_hbm.at[pl.ds(off, K)], idx_v)
            pltpu.async_copy(table_hbm.at[idx_v], rows_v, sem).wait()
            pltpu.sync_copy(rows_v, out_hbm.at[pl.ds(off, K)])
            return 0

        lax.fori_loop(0, ITERS, body, 0)

    return k(table, idx)


@functools.partial(jax.jit, static_argnames=("w",))
def _sc_scatter_add(rows, idx, zchunk, w):
    @functools.partial(
        pl.kernel,
        out_type=jax.ShapeDtypeStruct((NC, NN, w), jnp.float32),
        mesh=_mesh(),
        scratch_types=[
            pltpu.VMEM((K,), jnp.int32),
            pltpu.VMEM((K, w), jnp.float32),
            pltpu.VMEM_SHARED((NN, w), jnp.float32),
        ],
    )
    def k(rows_hbm, idx_hbm, z_hbm, out_hbm, idx_v, rows_v, acc_sh):
        cid = lax.axis_index("c")
        sid = lax.axis_index("s")
        base = (sid * NC + cid) * PER_W

        def zero(k, _):
            ch = sid * CPS + k

            @pl.when(ch < NCH)
            def _():
                pltpu.sync_copy(z_hbm, acc_sh.at[pl.ds(ch * WB, WB)])

            return 0

        lax.fori_loop(0, CPS, zero, 0)
        plsc.subcore_barrier()

        def body(i, _):
            off = base + i * K
            pltpu.sync_copy(idx_hbm.at[pl.ds(off, K)], idx_v)
            pltpu.sync_copy(rows_hbm.at[pl.ds(off, K)], rows_v)
            pltpu.sync_copy(rows_v, acc_sh.at[idx_v], add=True)
            return 0

        lax.fori_loop(0, ITERS, body, 0)
        plsc.subcore_barrier()

        def wout(k, _):
            ch = sid * CPS + k

            @pl.when(ch < NCH)
            def _():
                r0 = ch * WB
                pltpu.sync_copy(acc_sh.at[pl.ds(r0, WB)], out_hbm.at[cid, pl.ds(r0, WB)])

            return 0

        lax.fori_loop(0, CPS, wout, 0)

    return k(rows, idx, zchunk)


def _sig(v):
    return jax.nn.sigmoid(v)


def _full(shape, dtype=jnp.float32):
    return jax.ShapeDtypeStruct(shape, dtype)


def _tc_prep(x, Wn, bn, A0):
    def body(x_r, wn_r, bn_r, a0_r, nf_r, nfa_r):
        nf = jnp.dot(x_r[...], wn_r[...], preferred_element_type=jnp.float32) + bn_r[...]
        nf_r[...] = nf
        nfa_r[...] = jnp.dot(nf, a0_r[...], preferred_element_type=jnp.float32)

    return pl.pallas_call(
        body, out_shape=[_full((NN, H)), _full((NN, H))])(x, Wn, bn.reshape(1, H), A0)


def _tc_dist(qs, qd):
    def body(qs_r, qd_r, d_r):
        rel = qd_r[...] - qs_r[...]
        ssq = jnp.sum(rel * rel, axis=-1, keepdims=True)
        d_r[...] = jnp.sqrt(ssq + 1e-8)

    espec = pl.BlockSpec((BE, H), lambda i: (i, 0))
    return pl.pallas_call(
        body, grid=(GE,), in_specs=[espec, espec],
        out_specs=pl.BlockSpec((BE, 1), lambda i: (i, 0)),
        out_shape=_full((EE, 1)))(qs, qd)


def _tc_edge_fwd(ga, ea, dist, G, g0, c):
    def body(ga_r, ea_r, d_r, g_r, g0_r, c_r, msg_r, sigp_r):
        z = (ga_r[...] + jnp.dot(ea_r[...], g_r[...], preferred_element_type=jnp.float32)
             + g0_r[...] + d_r[...] * c_r[...])
        s = _sig(z)
        msg_r[...] = z * s
        sigp_r[...] = s * (1.0 + z * (1.0 - s))

    espec = pl.BlockSpec((BE, H), lambda i: (i, 0))
    return pl.pallas_call(
        body, grid=(GE,),
        in_specs=[espec,
                  pl.BlockSpec((BE, 16), lambda i: (i, 0)),
                  pl.BlockSpec((BE, 1), lambda i: (i, 0)),
                  pl.BlockSpec((16, H), lambda i: (0, 0)),
                  pl.BlockSpec((1, H), lambda i: (0, 0)),
                  pl.BlockSpec((1, H), lambda i: (0, 0))],
        out_specs=[espec, espec],
        out_shape=[_full((EE, H)), _full((EE, H))],
    )(ga, ea, dist, G, g0.reshape(1, H), c.reshape(1, H))


def _tc_node_fwd(nf, agg2, WuA, WuB, A_next):
    def body(nf_r, agg2_r, wua_r, wub_r, an_r, yln_r, istd_r, dsu_r, nfa_r):
        nf_ = nf_r[...]
        agg = agg2_r[0] + agg2_r[1]
        u = (jnp.dot(nf_, wua_r[...], preferred_element_type=jnp.float32)
             + jnp.dot(agg, wub_r[...], preferred_element_type=jnp.float32))
        s = _sig(u)
        upd = u * s
        dsu_r[...] = s * (1.0 + u * (1.0 - s))
        r = nf_ + upd
        m = jnp.mean(r, axis=-1, keepdims=True)
        cen = r - m
        var = jnp.mean(cen * cen, axis=-1, keepdims=True)
        istd = jax.lax.rsqrt(var + 1e-5)
        istd_r[...] = istd
        yln = cen * istd
        yln_r[...] = yln
        nfa_r[...] = jnp.dot(yln, an_r[...], preferred_element_type=jnp.float32)

    return pl.pallas_call(
        body,
        out_shape=[_full((NN, H)), _full((NN, 1)), _full((NN, H)), _full((NN, H))],
    )(nf, agg2, WuA, WuB, A_next)


def _tc_head(nf4, Wo, bo, Wp1, bp1, wp2row, Wp1T, WoT):
    def body(nf_r, wo_r, bo_r, wp1_r, bp1_r, wp2_r, wp1t_r, wot_r, dnf_r):
        out = jnp.dot(nf_r[...], wo_r[...], preferred_element_type=jnp.float32) + bo_r[...]
        o1 = jnp.dot(out, wp1_r[...], preferred_element_type=jnp.float32) + bp1_r[...]
        s = _sig(o1)
        do1 = wp2_r[...] * (s * (1.0 + o1 * (1.0 - s)))
        dout = jnp.dot(do1, wp1t_r[...], preferred_element_type=jnp.float32)
        dnf_r[...] = jnp.dot(dout, wot_r[...], preferred_element_type=jnp.float32)

    return pl.pallas_call(body, out_shape=_full((NN, H)))(
        nf4, Wo, bo.reshape(1, -1), Wp1, bp1.reshape(1, -1), wp2row, Wp1T, WoT)


def _tc_node_bwd(dnf, yln, istd, dsu, WuAT, WuBT):
    def body(dnf_r, yln_r, istd_r, dsu_r, wuat_r, wubt_r, dres_r, dagg_r):
        dnf_ = dnf_r[...]
        yln = yln_r[...]
        dr = istd_r[...] * (
            dnf_ - jnp.mean(dnf_, axis=-1, keepdims=True)
            - yln * jnp.mean(dnf_ * yln, axis=-1, keepdims=True))
        du = dr * dsu_r[...]
        dres_r[...] = dr + jnp.dot(du, wuat_r[...], preferred_element_type=jnp.float32)
        dagg_r[...] = jnp.dot(du, wubt_r[...], preferred_element_type=jnp.float32)

    return pl.pallas_call(
        body, out_shape=[_full((NN, H)), _full((NN, H))])(dnf, yln, istd, dsu, WuAT, WuBT)


def _tc_edge_bwd(gd, sigp, c, ddist_in):
    def body(gd_r, sigp_r, c_r, di_r, dz_r, do_r):
        dz = gd_r[...] * sigp_r[...]
        dz_r[...] = dz
        do_r[...] = di_r[...] + jnp.sum(dz * c_r[...], axis=-1, keepdims=True)

    espec = pl.BlockSpec((BE, H), lambda i: (i, 0))
    dspec = pl.BlockSpec((BE, 1), lambda i: (i, 0))
    return pl.pallas_call(
        body, grid=(GE,),
        in_specs=[espec, espec, pl.BlockSpec((1, H), lambda i: (0, 0)), dspec],
        out_specs=[espec, dspec],
        out_shape=[_full((EE, H)), _full((EE, 1))],
    )(gd, sigp, c.reshape(1, H), ddist_in)


def _tc_merge(dres, dnfa2, AT):
    def body(dres_r, dnfa2_r, at_r, dnf_r):
        dnfa = dnfa2_r[0] + dnfa2_r[1]
        dnf_r[...] = dres_r[...] + jnp.dot(dnfa, at_r[...], preferred_element_type=jnp.float32)

    return pl.pallas_call(body, out_shape=_full((NN, H)))(dres, dnfa2, AT)


def _tc_final_edge(qs, qd, ddist):
    def body(qs_r, qd_r, dd_r, drel_r):
        rel = qd_r[...] - qs_r[...]
        ssq = jnp.sum(rel * rel, axis=-1, keepdims=True)
        dist = jnp.sqrt(ssq + 1e-8)
        drel_r[...] = (dd_r[...] / dist) * rel

    espec = pl.BlockSpec((BE, H), lambda i: (i, 0))
    return pl.pallas_call(
        body, grid=(GE,),
        in_specs=[espec, espec, pl.BlockSpec((BE, 1), lambda i: (i, 0))],
        out_specs=espec, out_shape=_full((EE, H)))(qs, qd, ddist)


def _tc_finish(sdst, ssrc):
    def body(sd_r, ss_r, o_r):
        o_r[...] = ss_r[0] + ss_r[1] - sd_r[0] - sd_r[1]

    return pl.pallas_call(body, out_shape=_full((NN, H)))(sdst, ssrc)


def kernel(t, y, x, edge_index, edge_attr, Wn, bn, We, be, Wm, Wu, Wo, bo,
           Wp1, bp1, Wp2, bp2):
    src = edge_index[0].astype(jnp.int32)
    dst = edge_index[1].astype(jnp.int32)
    q = y[:, :3]
    p = y[:, 3:]
    q128 = jnp.pad(q, ((0, 0), (0, H - 3)))

    A = [Wm[l][:H] for l in range(LL)]
    G = [jnp.concatenate([Wm[l][H:H + 3], We @ Wm[l][H + 3:2 * H]], axis=0)
         for l in range(LL)]
    g0 = [be @ Wm[l][H + 3:2 * H] for l in range(LL)]
    c = [Wm[l][2 * H] for l in range(LL)]
    WuA = [Wu[l][:H] for l in range(LL)]
    WuB = [Wu[l][H:] for l in range(LL)]
    wp2row = jnp.broadcast_to(Wp2[:, 0], (1, Wp2.shape[0]))
    z128 = jnp.zeros((WB, H), jnp.float32)

    qs = _sc_gather(q128, src, H)
    qd = _sc_gather(q128, dst, H)
    dist = _tc_dist(qs, qd)

    nf, nfa = _tc_prep(x, Wn, bn, A[0])
    saves = []
    for l in range(LL):
        ga = _sc_gather(nfa, src, H)
        msg, sigp = _tc_edge_fwd(ga, edge_attr, dist, G[l], g0[l], c[l])
        agg2 = _sc_scatter_add(msg, dst, z128, H)
        A_next = A[l + 1] if l + 1 < LL else A[0]
        yln, istd, dsu, nfa = _tc_node_fwd(nf, agg2, WuA[l], WuB[l], A_next)
        saves.append((yln, istd, dsu, sigp))
        nf = yln

    dnf = _tc_head(nf, Wo, bo, Wp1, bp1, wp2row, Wp1.T, Wo.T)
    ddist = jnp.zeros((EE, 1), jnp.float32)
    for l in reversed(range(LL)):
        yln, istd, dsu, sigp = saves[l]
        dres, dagg = _tc_node_bwd(dnf, yln, istd, dsu, WuA[l].T, WuB[l].T)
        gd = _sc_gather(dagg, dst, H)
        dz, ddist = _tc_edge_bwd(gd, sigp, c[l], ddist)
        if l > 0:
            dnfa2 = _sc_scatter_add(dz, src, z128, H)
            dnf = _tc_merge(dres, dnfa2, A[l].T)

    drel = _tc_final_edge(qs, qd, ddist)
    sdst = _sc_scatter_add(drel, dst, z128, H)
    ssrc = _sc_scatter_add(drel, src, z128, H)
    gqneg = _tc_finish(sdst, ssrc)
    return jnp.concatenate([p, gqneg[:, :3]], axis=-1)


# speedup vs baseline: 2.9950x; 1.3776x over previous
import functools

import jax
import jax.numpy as jnp
from jax import lax
from jax.experimental import pallas as pl
from jax.experimental.pallas import tpu as pltpu
from jax.experimental.pallas import tpu_sc as plsc

NN = 10000
EE = 320000
H = 128
LL = 4

NC = 2
NS = 16
NW = NC * NS
PER_W = EE // NW
K = 80
ITERS = PER_W // K
WB = 80
NCH = NN // WB
CPS = -(-NCH // NS)
BE = 4000
GE = EE // BE


def _mesh():
    return plsc.VectorSubcoreMesh(core_axis_name="c", subcore_axis_name="s")


@functools.partial(jax.jit, static_argnames=("w",))
def _sc_gather(table, idx, w):
    @functools.partial(
        pl.kernel,
        out_type=jax.ShapeDtypeStruct((EE, w), jnp.float32),
        mesh=_mesh(),
        scratch_types=[
            pltpu.VMEM((2, K), jnp.int32),
            pltpu.VMEM((2, K, w), jnp.float32),
            pltpu.SemaphoreType.DMA,
            pltpu.SemaphoreType.DMA,
        ],
    )
    def k(table_hbm, idx_hbm, out_hbm, idx_v, rows_v, sem0, sem1):
        cid = lax.axis_index("c")
        sid = lax.axis_index("s")
        base = (sid * NC + cid) * PER_W
        sems = (sem0, sem1)

        pltpu.sync_copy(idx_hbm.at[pl.ds(base, K)], idx_v.at[0])
        pltpu.async_copy(table_hbm.at[idx_v.at[0]], rows_v.at[0], sems[0])

        def pair(pp, _):
            for b in range(2):
                i = pp * 2 + b

                @pl.when(i < ITERS)
                def _():
                    nb = 1 - b

                    @pl.when(i + 1 < ITERS)
                    def _():
                        noff = base + (i + 1) * K
                        pltpu.sync_copy(idx_hbm.at[pl.ds(noff, K)], idx_v.at[nb])
                        pltpu.async_copy(table_hbm.at[idx_v.at[nb]], rows_v.at[nb], sems[nb])

                    pltpu.make_async_copy(table_hbm.at[idx_v.at[b]], rows_v.at[b], sems[b]).wait()
                    pltpu.sync_copy(rows_v.at[b], out_hbm.at[pl.ds(base + i * K, K)])

            return 0

        lax.fori_loop(0, (ITERS + 1) // 2, pair, 0)

    return k(table, idx)


@functools.partial(jax.jit, static_argnames=("w",))
def _sc_scatter_add(rows, idx, zchunk, w):
    @functools.partial(
        pl.kernel,
        out_type=jax.ShapeDtypeStruct((NC, NN, w), jnp.float32),
        mesh=_mesh(),
        scratch_types=[
            pltpu.VMEM((2, K), jnp.int32),
            pltpu.VMEM((2, K, w), jnp.float32),
            pltpu.VMEM_SHARED((NN, w), jnp.float32),
            pltpu.SemaphoreType.DMA,
            pltpu.SemaphoreType.DMA,
        ],
    )
    def k(rows_hbm, idx_hbm, z_hbm, out_hbm, idx_v, rows_v, acc_sh, sem0, sem1):
        cid = lax.axis_index("c")
        sid = lax.axis_index("s")
        base = (sid * NC + cid) * PER_W
        sems = (sem0, sem1)

        def zero(k, _):
            ch = sid * CPS + k

            @pl.when(ch < NCH)
            def _():
                pltpu.sync_copy(z_hbm, acc_sh.at[pl.ds(ch * WB, WB)])

            return 0

        lax.fori_loop(0, CPS, zero, 0)
        plsc.subcore_barrier()

        pltpu.sync_copy(idx_hbm.at[pl.ds(base, K)], idx_v.at[0])
        pltpu.async_copy(rows_hbm.at[pl.ds(base, K)], rows_v.at[0], sems[0])

        def pair(pp, _):
            for b in range(2):
                i = pp * 2 + b

                @pl.when(i < ITERS)
                def _():
                    nb = 1 - b

                    @pl.when(i + 1 < ITERS)
                    def _():
                        noff = base + (i + 1) * K
                        pltpu.sync_copy(idx_hbm.at[pl.ds(noff, K)], idx_v.at[nb])
                        pltpu.async_copy(rows_hbm.at[pl.ds(noff, K)], rows_v.at[nb], sems[nb])

                    pltpu.make_async_copy(
                        rows_hbm.at[pl.ds(base + i * K, K)], rows_v.at[b], sems[b]).wait()
                    pltpu.sync_copy(rows_v.at[b], acc_sh.at[idx_v.at[b]], add=True)

            return 0

        lax.fori_loop(0, (ITERS + 1) // 2, pair, 0)
        plsc.subcore_barrier()

        def wout(k, _):
            ch = sid * CPS + k

            @pl.when(ch < NCH)
            def _():
                r0 = ch * WB
                pltpu.sync_copy(acc_sh.at[pl.ds(r0, WB)], out_hbm.at[cid, pl.ds(r0, WB)])

            return 0

        lax.fori_loop(0, CPS, wout, 0)

    return k(rows, idx, zchunk)


def _sig(v):
    return jax.nn.sigmoid(v)


def _full(shape, dtype=jnp.float32):
    return jax.ShapeDtypeStruct(shape, dtype)


def _tc_prep(x, Wn, bn, A0):
    def body(x_r, wn_r, bn_r, a0_r, nf_r, nfa_r):
        nf = jnp.dot(x_r[...], wn_r[...], preferred_element_type=jnp.float32) + bn_r[...]
        nf_r[...] = nf
        nfa_r[...] = jnp.dot(nf, a0_r[...], preferred_element_type=jnp.float32)

    return pl.pallas_call(
        body, out_shape=[_full((NN, H)), _full((NN, H))])(x, Wn, bn.reshape(1, H), A0)


def _tc_dist(qs, qd):
    def body(qs_r, qd_r, d_r):
        rel = qd_r[...] - qs_r[...]
        ssq = jnp.sum(rel * rel, axis=-1, keepdims=True)
        d_r[...] = jnp.sqrt(ssq + 1e-8)

    espec = pl.BlockSpec((BE, H), lambda i: (i, 0))
    return pl.pallas_call(
        body, grid=(GE,), in_specs=[espec, espec],
        out_specs=pl.BlockSpec((BE, 1), lambda i: (i, 0)),
        out_shape=_full((EE, 1)))(qs, qd)


def _tc_edge_fwd(ga, ea, dist, G, g0, c):
    def body(ga_r, ea_r, d_r, g_r, g0_r, c_r, msg_r, sigp_r):
        z = (ga_r[...] + jnp.dot(ea_r[...], g_r[...], preferred_element_type=jnp.float32)
             + g0_r[...] + d_r[...] * c_r[...])
        s = _sig(z)
        msg_r[...] = z * s
        sigp_r[...] = s * (1.0 + z * (1.0 - s))

    espec = pl.BlockSpec((BE, H), lambda i: (i, 0))
    return pl.pallas_call(
        body, grid=(GE,),
        in_specs=[espec,
                  pl.BlockSpec((BE, 16), lambda i: (i, 0)),
                  pl.BlockSpec((BE, 1), lambda i: (i, 0)),
                  pl.BlockSpec((16, H), lambda i: (0, 0)),
                  pl.BlockSpec((1, H), lambda i: (0, 0)),
                  pl.BlockSpec((1, H), lambda i: (0, 0))],
        out_specs=[espec, espec],
        out_shape=[_full((EE, H)), _full((EE, H))],
    )(ga, ea, dist, G, g0.reshape(1, H), c.reshape(1, H))


def _tc_node_fwd(nf, agg2, WuA, WuB, A_next):
    def body(nf_r, agg2_r, wua_r, wub_r, an_r, yln_r, istd_r, dsu_r, nfa_r):
        nf_ = nf_r[...]
        agg = agg2_r[0] + agg2_r[1]
        u = (jnp.dot(nf_, wua_r[...], preferred_element_type=jnp.float32)
             + jnp.dot(agg, wub_r[...], preferred_element_type=jnp.float32))
        s = _sig(u)
        upd = u * s
        dsu_r[...] = s * (1.0 + u * (1.0 - s))
        r = nf_ + upd
        m = jnp.mean(r, axis=-1, keepdims=True)
        cen = r - m
        var = jnp.mean(cen * cen, axis=-1, keepdims=True)
        istd = jax.lax.rsqrt(var + 1e-5)
        istd_r[...] = istd
        yln = cen * istd
        yln_r[...] = yln
        nfa_r[...] = jnp.dot(yln, an_r[...], preferred_element_type=jnp.float32)

    return pl.pallas_call(
        body,
        out_shape=[_full((NN, H)), _full((NN, 1)), _full((NN, H)), _full((NN, H))],
    )(nf, agg2, WuA, WuB, A_next)


def _tc_head(nf4, Wo, bo, Wp1, bp1, wp2row, Wp1T, WoT):
    def body(nf_r, wo_r, bo_r, wp1_r, bp1_r, wp2_r, wp1t_r, wot_r, dnf_r):
        out = jnp.dot(nf_r[...], wo_r[...], preferred_element_type=jnp.float32) + bo_r[...]
        o1 = jnp.dot(out, wp1_r[...], preferred_element_type=jnp.float32) + bp1_r[...]
        s = _sig(o1)
        do1 = wp2_r[...] * (s * (1.0 + o1 * (1.0 - s)))
        dout = jnp.dot(do1, wp1t_r[...], preferred_element_type=jnp.float32)
        dnf_r[...] = jnp.dot(dout, wot_r[...], preferred_element_type=jnp.float32)

    return pl.pallas_call(body, out_shape=_full((NN, H)))(
        nf4, Wo, bo.reshape(1, -1), Wp1, bp1.reshape(1, -1), wp2row, Wp1T, WoT)


def _tc_node_bwd(dnf, yln, istd, dsu, WuAT, WuBT):
    def body(dnf_r, yln_r, istd_r, dsu_r, wuat_r, wubt_r, dres_r, dagg_r):
        dnf_ = dnf_r[...]
        yln = yln_r[...]
        dr = istd_r[...] * (
            dnf_ - jnp.mean(dnf_, axis=-1, keepdims=True)
            - yln * jnp.mean(dnf_ * yln, axis=-1, keepdims=True))
        du = dr * dsu_r[...]
        dres_r[...] = dr + jnp.dot(du, wuat_r[...], preferred_element_type=jnp.float32)
        dagg_r[...] = jnp.dot(du, wubt_r[...], preferred_element_type=jnp.float32)

    return pl.pallas_call(
        body, out_shape=[_full((NN, H)), _full((NN, H))])(dnf, yln, istd, dsu, WuAT, WuBT)


def _tc_edge_bwd(gd, sigp, c, ddist_in):
    def body(gd_r, sigp_r, c_r, di_r, dz_r, do_r):
        dz = gd_r[...] * sigp_r[...]
        dz_r[...] = dz
        do_r[...] = di_r[...] + jnp.sum(dz * c_r[...], axis=-1, keepdims=True)

    espec = pl.BlockSpec((BE, H), lambda i: (i, 0))
    dspec = pl.BlockSpec((BE, 1), lambda i: (i, 0))
    return pl.pallas_call(
        body, grid=(GE,),
        in_specs=[espec, espec, pl.BlockSpec((1, H), lambda i: (0, 0)), dspec],
        out_specs=[espec, dspec],
        out_shape=[_full((EE, H)), _full((EE, 1))],
    )(gd, sigp, c.reshape(1, H), ddist_in)


def _tc_merge(dres, dnfa2, AT):
    def body(dres_r, dnfa2_r, at_r, dnf_r):
        dnfa = dnfa2_r[0] + dnfa2_r[1]
        dnf_r[...] = dres_r[...] + jnp.dot(dnfa, at_r[...], preferred_element_type=jnp.float32)

    return pl.pallas_call(body, out_shape=_full((NN, H)))(dres, dnfa2, AT)


def _tc_final_edge(qs, qd, ddist):
    def body(qs_r, qd_r, dd_r, drel_r):
        rel = qd_r[...] - qs_r[...]
        ssq = jnp.sum(rel * rel, axis=-1, keepdims=True)
        dist = jnp.sqrt(ssq + 1e-8)
        drel_r[...] = (dd_r[...] / dist) * rel

    espec = pl.BlockSpec((BE, H), lambda i: (i, 0))
    return pl.pallas_call(
        body, grid=(GE,),
        in_specs=[espec, espec, pl.BlockSpec((BE, 1), lambda i: (i, 0))],
        out_specs=espec, out_shape=_full((EE, H)))(qs, qd, ddist)


def _tc_finish(sdst, ssrc):
    def body(sd_r, ss_r, o_r):
        o_r[...] = ss_r[0] + ss_r[1] - sd_r[0] - sd_r[1]

    return pl.pallas_call(body, out_shape=_full((NN, H)))(sdst, ssrc)


def kernel(t, y, x, edge_index, edge_attr, Wn, bn, We, be, Wm, Wu, Wo, bo,
           Wp1, bp1, Wp2, bp2):
    src = edge_index[0].astype(jnp.int32)
    dst = edge_index[1].astype(jnp.int32)
    q = y[:, :3]
    p = y[:, 3:]
    q128 = jnp.pad(q, ((0, 0), (0, H - 3)))

    A = [Wm[l][:H] for l in range(LL)]
    G = [jnp.concatenate([Wm[l][H:H + 3], We @ Wm[l][H + 3:2 * H]], axis=0)
         for l in range(LL)]
    g0 = [be @ Wm[l][H + 3:2 * H] for l in range(LL)]
    c = [Wm[l][2 * H] for l in range(LL)]
    WuA = [Wu[l][:H] for l in range(LL)]
    WuB = [Wu[l][H:] for l in range(LL)]
    wp2row = jnp.broadcast_to(Wp2[:, 0], (1, Wp2.shape[0]))
    z128 = jnp.zeros((WB, H), jnp.float32)

    qs = _sc_gather(q128, src, H)
    qd = _sc_gather(q128, dst, H)
    dist = _tc_dist(qs, qd)

    nf, nfa = _tc_prep(x, Wn, bn, A[0])
    saves = []
    for l in range(LL):
        ga = _sc_gather(nfa, src, H)
        msg, sigp = _tc_edge_fwd(ga, edge_attr, dist, G[l], g0[l], c[l])
        agg2 = _sc_scatter_add(msg, dst, z128, H)
        A_next = A[l + 1] if l + 1 < LL else A[0]
        yln, istd, dsu, nfa = _tc_node_fwd(nf, agg2, WuA[l], WuB[l], A_next)
        saves.append((yln, istd, dsu, sigp))
        nf = yln

    dnf = _tc_head(nf, Wo, bo, Wp1, bp1, wp2row, Wp1.T, Wo.T)
    ddist = jnp.zeros((EE, 1), jnp.float32)
    for l in reversed(range(LL)):
        yln, istd, dsu, sigp = saves[l]
        dres, dagg = _tc_node_bwd(dnf, yln, istd, dsu, WuA[l].T, WuB[l].T)
        gd = _sc_gather(dagg, dst, H)
        dz, ddist = _tc_edge_bwd(gd, sigp, c[l], ddist)
        if l > 0:
            dnfa2 = _sc_scatter_add(dz, src, z128, H)
            dnf = _tc_merge(dres, dnfa2, A[l].T)

    drel = _tc_final_edge(qs, qd, ddist)
    sdst = _sc_scatter_add(drel, dst, z128, H)
    ssrc = _sc_scatter_add(drel, src, z128, H)
    gqneg = _tc_finish(sdst, ssrc)
    return jnp.concatenate([p, gqneg[:, :3]], axis=-1)


# speedup vs baseline: 3.1601x; 1.0551x over previous
import functools

import jax
import jax.numpy as jnp
from jax import lax
from jax.experimental import pallas as pl
from jax.experimental.pallas import tpu as pltpu
from jax.experimental.pallas import tpu_sc as plsc

NN = 10000
EE = 320000
H = 128
LL = 4

NC = 2
NS = 16
NW = NC * NS
PER_W = EE // NW
K = 80
ITERS = PER_W // K
WB = 80
NCH = NN // WB
CPS = -(-NCH // NS)
BE = 4000
GE = EE // BE


def _mesh():
    return plsc.VectorSubcoreMesh(core_axis_name="c", subcore_axis_name="s")


@functools.partial(jax.jit, static_argnames=("w",))
def _sc_gather(table, idx, w):
    @functools.partial(
        pl.kernel,
        out_type=jax.ShapeDtypeStruct((EE, w), jnp.float32),
        mesh=_mesh(),
        scratch_types=[
            pltpu.VMEM((2, K), jnp.int32),
            pltpu.VMEM((2, K, w), jnp.float32),
            pltpu.SemaphoreType.DMA,
            pltpu.SemaphoreType.DMA,
            pltpu.SemaphoreType.DMA,
            pltpu.SemaphoreType.DMA,
        ],
    )
    def k(table_hbm, idx_hbm, out_hbm, idx_v, rows_v, sem0, sem1, wsem0, wsem1):
        cid = lax.axis_index("c")
        sid = lax.axis_index("s")
        base = (sid * NC + cid) * PER_W
        sems = (sem0, sem1)
        wsems = (wsem0, wsem1)

        pltpu.sync_copy(idx_hbm.at[pl.ds(base, K)], idx_v.at[0])
        pltpu.async_copy(table_hbm.at[idx_v.at[0]], rows_v.at[0], sems[0])

        def pair(pp, _):
            for b in range(2):
                i = pp * 2 + b

                @pl.when(i < ITERS)
                def _():
                    nb = 1 - b

                    @pl.when(i + 1 < ITERS)
                    def _():
                        noff = base + (i + 1) * K
                        pltpu.sync_copy(idx_hbm.at[pl.ds(noff, K)], idx_v.at[nb])

                        @pl.when(i >= 1)
                        def _():
                            pltpu.make_async_copy(
                                rows_v.at[nb], out_hbm.at[pl.ds(base, K)], wsems[nb]).wait()

                        pltpu.async_copy(table_hbm.at[idx_v.at[nb]], rows_v.at[nb], sems[nb])

                    pltpu.make_async_copy(table_hbm.at[idx_v.at[b]], rows_v.at[b], sems[b]).wait()
                    pltpu.async_copy(rows_v.at[b], out_hbm.at[pl.ds(base + i * K, K)], wsems[b])

            return 0

        lax.fori_loop(0, (ITERS + 1) // 2, pair, 0)
        pltpu.make_async_copy(rows_v.at[0], out_hbm.at[pl.ds(base, K)], wsems[0]).wait()
        pltpu.make_async_copy(rows_v.at[1], out_hbm.at[pl.ds(base, K)], wsems[1]).wait()

    return k(table, idx)


@functools.partial(jax.jit, static_argnames=("w",))
def _sc_scatter_add(rows, idx, zchunk, w):
    @functools.partial(
        pl.kernel,
        out_type=jax.ShapeDtypeStruct((NC, NN, w), jnp.float32),
        mesh=_mesh(),
        scratch_types=[
            pltpu.VMEM((2, K), jnp.int32),
            pltpu.VMEM((2, K, w), jnp.float32),
            pltpu.VMEM_SHARED((NN, w), jnp.float32),
            pltpu.SemaphoreType.DMA,
            pltpu.SemaphoreType.DMA,
        ],
    )
    def k(rows_hbm, idx_hbm, z_hbm, out_hbm, idx_v, rows_v, acc_sh, sem0, sem1):
        cid = lax.axis_index("c")
        sid = lax.axis_index("s")
        base = (sid * NC + cid) * PER_W
        sems = (sem0, sem1)

        def zero(k, _):
            ch = sid * CPS + k

            @pl.when(ch < NCH)
            def _():
                pltpu.sync_copy(z_hbm, acc_sh.at[pl.ds(ch * WB, WB)])

            return 0

        lax.fori_loop(0, CPS, zero, 0)
        plsc.subcore_barrier()

        pltpu.sync_copy(idx_hbm.at[pl.ds(base, K)], idx_v.at[0])
        pltpu.async_copy(rows_hbm.at[pl.ds(base, K)], rows_v.at[0], sems[0])

        def pair(pp, _):
            for b in range(2):
                i = pp * 2 + b

                @pl.when(i < ITERS)
                def _():
                    nb = 1 - b

                    @pl.when(i + 1 < ITERS)
                    def _():
                        noff = base + (i + 1) * K
                        pltpu.sync_copy(idx_hbm.at[pl.ds(noff, K)], idx_v.at[nb])
                        pltpu.async_copy(rows_hbm.at[pl.ds(noff, K)], rows_v.at[nb], sems[nb])

                    pltpu.make_async_copy(
                        rows_hbm.at[pl.ds(base + i * K, K)], rows_v.at[b], sems[b]).wait()
                    pltpu.sync_copy(rows_v.at[b], acc_sh.at[idx_v.at[b]], add=True)

            return 0

        lax.fori_loop(0, (ITERS + 1) // 2, pair, 0)
        plsc.subcore_barrier()

        def wout(k, _):
            ch = sid * CPS + k

            @pl.when(ch < NCH)
            def _():
                r0 = ch * WB
                pltpu.sync_copy(acc_sh.at[pl.ds(r0, WB)], out_hbm.at[cid, pl.ds(r0, WB)])

            return 0

        lax.fori_loop(0, CPS, wout, 0)

    return k(rows, idx, zchunk)


@jax.jit
def _sc_rel(q128, src, dst):
    @functools.partial(
        pl.kernel,
        out_type=jax.ShapeDtypeStruct((EE, H), jnp.float32),
        mesh=_mesh(),
        scratch_types=[
            pltpu.VMEM((2, K), jnp.int32),
            pltpu.VMEM((2, K), jnp.int32),
            pltpu.VMEM((2, K, H), jnp.float32),
            pltpu.VMEM((2, K, H), jnp.float32),
            pltpu.SemaphoreType.DMA,
            pltpu.SemaphoreType.DMA,
            pltpu.SemaphoreType.DMA,
            pltpu.SemaphoreType.DMA,
            pltpu.SemaphoreType.DMA,
            pltpu.SemaphoreType.DMA,
        ],
    )
    def k(q_hbm, src_hbm, dst_hbm, out_hbm, ixs_v, ixd_v, qs_v, qd_v,
          ss0, ss1, sd0, sd1, ws0, ws1):
        cid = lax.axis_index("c")
        sid = lax.axis_index("s")
        base = (sid * NC + cid) * PER_W
        ssems = (ss0, ss1)
        dsems = (sd0, sd1)
        wsems = (ws0, ws1)

        def start(i, b):
            off = base + i * K
            pltpu.sync_copy(src_hbm.at[pl.ds(off, K)], ixs_v.at[b])
            pltpu.sync_copy(dst_hbm.at[pl.ds(off, K)], ixd_v.at[b])
            pltpu.async_copy(q_hbm.at[ixs_v.at[b]], qs_v.at[b], ssems[b])
            pltpu.async_copy(q_hbm.at[ixd_v.at[b]], qd_v.at[b], dsems[b])

        start(0, 0)

        def pair(pp, _):
            for b in range(2):
                i = pp * 2 + b

                @pl.when(i < ITERS)
                def _():
                    nb = 1 - b

                    @pl.when(i + 1 < ITERS)
                    def _():
                        @pl.when(i >= 1)
                        def _():
                            pltpu.make_async_copy(
                                qd_v.at[nb], out_hbm.at[pl.ds(base, K)], wsems[nb]).wait()

                        start(i + 1, nb)

                    pltpu.make_async_copy(q_hbm.at[ixs_v.at[b]], qs_v.at[b], ssems[b]).wait()
                    pltpu.make_async_copy(q_hbm.at[ixd_v.at[b]], qd_v.at[b], dsems[b]).wait()

                    def sub(r, _):
                        qd_v[b, r, pl.ds(0, 16)] = (qd_v[b, r, pl.ds(0, 16)]
                                                    - qs_v[b, r, pl.ds(0, 16)])
                        return 0

                    lax.fori_loop(0, K, sub, 0)
                    pltpu.async_copy(qd_v.at[b], out_hbm.at[pl.ds(base + i * K, K)], wsems[b])

            return 0

        lax.fori_loop(0, (ITERS + 1) // 2, pair, 0)
        pltpu.make_async_copy(qd_v.at[0], out_hbm.at[pl.ds(base, K)], wsems[0]).wait()
        pltpu.make_async_copy(qd_v.at[1], out_hbm.at[pl.ds(base, K)], wsems[1]).wait()

    return k(q128, src, dst)


@jax.jit
def _sc_scatter_pm(rows, src, dst, zchunk):
    @functools.partial(
        pl.kernel,
        out_type=jax.ShapeDtypeStruct((NC, NN, H), jnp.float32),
        mesh=_mesh(),
        scratch_types=[
            pltpu.VMEM((2, K), jnp.int32),
            pltpu.VMEM((2, K), jnp.int32),
            pltpu.VMEM((2, K, H), jnp.float32),
            pltpu.VMEM_SHARED((NN, H), jnp.float32),
            pltpu.SemaphoreType.DMA,
            pltpu.SemaphoreType.DMA,
        ],
    )
    def k(rows_hbm, src_hbm, dst_hbm, z_hbm, out_hbm, ixs_v, ixd_v, rows_v,
          acc_sh, sem0, sem1):
        cid = lax.axis_index("c")
        sid = lax.axis_index("s")
        base = (sid * NC + cid) * PER_W
        sems = (sem0, sem1)

        def zero(kk, _):
            ch = sid * CPS + kk

            @pl.when(ch < NCH)
            def _():
                pltpu.sync_copy(z_hbm, acc_sh.at[pl.ds(ch * WB, WB)])

            return 0

        lax.fori_loop(0, CPS, zero, 0)
        plsc.subcore_barrier()

        pltpu.sync_copy(src_hbm.at[pl.ds(base, K)], ixs_v.at[0])
        pltpu.sync_copy(dst_hbm.at[pl.ds(base, K)], ixd_v.at[0])
        pltpu.async_copy(rows_hbm.at[pl.ds(base, K)], rows_v.at[0], sems[0])

        def pair(pp, _):
            for b in range(2):
                i = pp * 2 + b

                @pl.when(i < ITERS)
                def _():
                    nb = 1 - b

                    @pl.when(i + 1 < ITERS)
                    def _():
                        noff = base + (i + 1) * K
                        pltpu.sync_copy(src_hbm.at[pl.ds(noff, K)], ixs_v.at[nb])
                        pltpu.sync_copy(dst_hbm.at[pl.ds(noff, K)], ixd_v.at[nb])
                        pltpu.async_copy(rows_hbm.at[pl.ds(noff, K)], rows_v.at[nb], sems[nb])

                    pltpu.make_async_copy(
                        rows_hbm.at[pl.ds(base + i * K, K)], rows_v.at[b], sems[b]).wait()
                    pltpu.sync_copy(rows_v.at[b], acc_sh.at[ixs_v.at[b]], add=True)

                    def neg(r, _):
                        rows_v[b, r, pl.ds(0, 16)] = -rows_v[b, r, pl.ds(0, 16)]
                        return 0

                    lax.fori_loop(0, K, neg, 0)
                    pltpu.sync_copy(rows_v.at[b], acc_sh.at[ixd_v.at[b]], add=True)

            return 0

        lax.fori_loop(0, (ITERS + 1) // 2, pair, 0)
        plsc.subcore_barrier()

        def wout(kk, _):
            ch = sid * CPS + kk

            @pl.when(ch < NCH)
            def _():
                r0 = ch * WB
                pltpu.sync_copy(acc_sh.at[pl.ds(r0, WB)], out_hbm.at[cid, pl.ds(r0, WB)])

            return 0

        lax.fori_loop(0, CPS, wout, 0)

    return k(rows, src, dst, zchunk)


def _sig(v):
    return jax.nn.sigmoid(v)


def _full(shape, dtype=jnp.float32):
    return jax.ShapeDtypeStruct(shape, dtype)


def _tc_prep(x, Wn, bn, A0):
    def body(x_r, wn_r, bn_r, a0_r, nf_r, nfa_r):
        nf = jnp.dot(x_r[...], wn_r[...], preferred_element_type=jnp.float32) + bn_r[...]
        nf_r[...] = nf
        nfa_r[...] = jnp.dot(nf, a0_r[...], preferred_element_type=jnp.float32)

    return pl.pallas_call(
        body, out_shape=[_full((NN, H)), _full((NN, H))])(x, Wn, bn.reshape(1, H), A0)


def _tc_dist(rel):
    def body(rel_r, d_r):
        rel_ = rel_r[...]
        ssq = jnp.sum(rel_ * rel_, axis=-1, keepdims=True)
        d_r[...] = jnp.sqrt(ssq + 1e-8)

    espec = pl.BlockSpec((BE, H), lambda i: (i, 0))
    return pl.pallas_call(
        body, grid=(GE,), in_specs=[espec],
        out_specs=pl.BlockSpec((BE, 1), lambda i: (i, 0)),
        out_shape=_full((EE, 1)))(rel)


def _tc_edge_fwd(ga, ea, dist, G, g0, c):
    def body(ga_r, ea_r, d_r, g_r, g0_r, c_r, msg_r, sigp_r):
        z = (ga_r[...] + jnp.dot(ea_r[...], g_r[...], preferred_element_type=jnp.float32)
             + g0_r[...] + d_r[...] * c_r[...])
        s = _sig(z)
        msg_r[...] = z * s
        sigp_r[...] = s * (1.0 + z * (1.0 - s))

    espec = pl.BlockSpec((BE, H), lambda i: (i, 0))
    return pl.pallas_call(
        body, grid=(GE,),
        in_specs=[espec,
                  pl.BlockSpec((BE, 16), lambda i: (i, 0)),
                  pl.BlockSpec((BE, 1), lambda i: (i, 0)),
                  pl.BlockSpec((16, H), lambda i: (0, 0)),
                  pl.BlockSpec((1, H), lambda i: (0, 0)),
                  pl.BlockSpec((1, H), lambda i: (0, 0))],
        out_specs=[espec, espec],
        out_shape=[_full((EE, H)), _full((EE, H))],
    )(ga, ea, dist, G, g0.reshape(1, H), c.reshape(1, H))


def _tc_node_fwd(nf, agg2, WuA, WuB, A_next):
    def body(nf_r, agg2_r, wua_r, wub_r, an_r, yln_r, istd_r, dsu_r, nfa_r):
        nf_ = nf_r[...]
        agg = agg2_r[0] + agg2_r[1]
        u = (jnp.dot(nf_, wua_r[...], preferred_element_type=jnp.float32)
             + jnp.dot(agg, wub_r[...], preferred_element_type=jnp.float32))
        s = _sig(u)
        upd = u * s
        dsu_r[...] = s * (1.0 + u * (1.0 - s))
        r = nf_ + upd
        m = jnp.mean(r, axis=-1, keepdims=True)
        cen = r - m
        var = jnp.mean(cen * cen, axis=-1, keepdims=True)
        istd = jax.lax.rsqrt(var + 1e-5)
        istd_r[...] = istd
        yln = cen * istd
        yln_r[...] = yln
        nfa_r[...] = jnp.dot(yln, an_r[...], preferred_element_type=jnp.float32)

    return pl.pallas_call(
        body,
        out_shape=[_full((NN, H)), _full((NN, 1)), _full((NN, H)), _full((NN, H))],
    )(nf, agg2, WuA, WuB, A_next)


def _tc_head(nf4, Wo, bo, Wp1, bp1, wp2row, Wp1T, WoT):
    def body(nf_r, wo_r, bo_r, wp1_r, bp1_r, wp2_r, wp1t_r, wot_r, dnf_r):
        out = jnp.dot(nf_r[...], wo_r[...], preferred_element_type=jnp.float32) + bo_r[...]
        o1 = jnp.dot(out, wp1_r[...], preferred_element_type=jnp.float32) + bp1_r[...]
        s = _sig(o1)
        do1 = wp2_r[...] * (s * (1.0 + o1 * (1.0 - s)))
        dout = jnp.dot(do1, wp1t_r[...], preferred_element_type=jnp.float32)
        dnf_r[...] = jnp.dot(dout, wot_r[...], preferred_element_type=jnp.float32)

    return pl.pallas_call(body, out_shape=_full((NN, H)))(
        nf4, Wo, bo.reshape(1, -1), Wp1, bp1.reshape(1, -1), wp2row, Wp1T, WoT)


def _tc_node_bwd(dnf, yln, istd, dsu, WuAT, WuBT):
    def body(dnf_r, yln_r, istd_r, dsu_r, wuat_r, wubt_r, dres_r, dagg_r):
        dnf_ = dnf_r[...]
        yln = yln_r[...]
        dr = istd_r[...] * (
            dnf_ - jnp.mean(dnf_, axis=-1, keepdims=True)
            - yln * jnp.mean(dnf_ * yln, axis=-1, keepdims=True))
        du = dr * dsu_r[...]
        dres_r[...] = dr + jnp.dot(du, wuat_r[...], preferred_element_type=jnp.float32)
        dagg_r[...] = jnp.dot(du, wubt_r[...], preferred_element_type=jnp.float32)

    return pl.pallas_call(
        body, out_shape=[_full((NN, H)), _full((NN, H))])(dnf, yln, istd, dsu, WuAT, WuBT)


def _tc_edge_bwd(gd, sigp, c, ddist_in):
    def body(gd_r, sigp_r, c_r, di_r, dz_r, do_r):
        dz = gd_r[...] * sigp_r[...]
        dz_r[...] = dz
        do_r[...] = di_r[...] + jnp.sum(dz * c_r[...], axis=-1, keepdims=True)

    espec = pl.BlockSpec((BE, H), lambda i: (i, 0))
    dspec = pl.BlockSpec((BE, 1), lambda i: (i, 0))
    return pl.pallas_call(
        body, grid=(GE,),
        in_specs=[espec, espec, pl.BlockSpec((1, H), lambda i: (0, 0)), dspec],
        out_specs=[espec, dspec],
        out_shape=[_full((EE, H)), _full((EE, 1))],
    )(gd, sigp, c.reshape(1, H), ddist_in)


def _tc_merge(dres, dnfa2, AT):
    def body(dres_r, dnfa2_r, at_r, dnf_r):
        dnfa = dnfa2_r[0] + dnfa2_r[1]
        dnf_r[...] = dres_r[...] + jnp.dot(dnfa, at_r[...], preferred_element_type=jnp.float32)

    return pl.pallas_call(body, out_shape=_full((NN, H)))(dres, dnfa2, AT)


def _tc_final_edge(rel, ddist):
    def body(rel_r, dd_r, drel_r):
        rel_ = rel_r[...]
        ssq = jnp.sum(rel_ * rel_, axis=-1, keepdims=True)
        dist = jnp.sqrt(ssq + 1e-8)
        drel_r[...] = (dd_r[...] / dist) * rel_

    espec = pl.BlockSpec((BE, H), lambda i: (i, 0))
    return pl.pallas_call(
        body, grid=(GE,),
        in_specs=[espec, pl.BlockSpec((BE, 1), lambda i: (i, 0))],
        out_specs=espec, out_shape=_full((EE, H)))(rel, ddist)


def _tc_finish(spm):
    def body(s_r, o_r):
        o_r[...] = s_r[0] + s_r[1]

    return pl.pallas_call(body, out_shape=_full((NN, H)))(spm)


def kernel(t, y, x, edge_index, edge_attr, Wn, bn, We, be, Wm, Wu, Wo, bo,
           Wp1, bp1, Wp2, bp2):
    src = edge_index[0].astype(jnp.int32)
    dst = edge_index[1].astype(jnp.int32)
    q = y[:, :3]
    p = y[:, 3:]
    q128 = jnp.pad(q, ((0, 0), (0, H - 3)))

    A = [Wm[l][:H] for l in range(LL)]
    G = [jnp.concatenate([Wm[l][H:H + 3], We @ Wm[l][H + 3:2 * H]], axis=0)
         for l in range(LL)]
    g0 = [be @ Wm[l][H + 3:2 * H] for l in range(LL)]
    c = [Wm[l][2 * H] for l in range(LL)]
    WuA = [Wu[l][:H] for l in range(LL)]
    WuB = [Wu[l][H:] for l in range(LL)]
    wp2row = jnp.broadcast_to(Wp2[:, 0], (1, Wp2.shape[0]))
    z128 = jnp.zeros((WB, H), jnp.float32)

    rel = _sc_rel(q128, src, dst)
    dist = _tc_dist(rel)

    nf, nfa = _tc_prep(x, Wn, bn, A[0])
    saves = []
    for l in range(LL):
        ga = _sc_gather(nfa, src, H)
        msg, sigp = _tc_edge_fwd(ga, edge_attr, dist, G[l], g0[l], c[l])
        agg2 = _sc_scatter_add(msg, dst, z128, H)
        A_next = A[l + 1] if l + 1 < LL else A[0]
        yln, istd, dsu, nfa = _tc_node_fwd(nf, agg2, WuA[l], WuB[l], A_next)
        saves.append((yln, istd, dsu, sigp))
        nf = yln

    dnf = _tc_head(nf, Wo, bo, Wp1, bp1, wp2row, Wp1.T, Wo.T)
    ddist = jnp.zeros((EE, 1), jnp.float32)
    for l in reversed(range(LL)):
        yln, istd, dsu, sigp = saves[l]
        dres, dagg = _tc_node_bwd(dnf, yln, istd, dsu, WuA[l].T, WuB[l].T)
        gd = _sc_gather(dagg, dst, H)
        dz, ddist = _tc_edge_bwd(gd, sigp, c[l], ddist)
        if l > 0:
            dnfa2 = _sc_scatter_add(dz, src, z128, H)
            dnf = _tc_merge(dres, dnfa2, A[l].T)

    drel = _tc_final_edge(rel, ddist)
    spm = _sc_scatter_pm(drel, src, dst, z128)
    gqneg = _tc_finish(spm)
    return jnp.concatenate([p, gqneg[:, :3]], axis=-1)


# speedup vs baseline: 3.3898x; 1.0727x over previous
import functools

import jax
import jax.numpy as jnp
from jax import lax
from jax.experimental import pallas as pl
from jax.experimental.pallas import tpu as pltpu
from jax.experimental.pallas import tpu_sc as plsc

NN = 10000
EE = 320000
H = 128
LL = 4

NC = 2
NS = 16
NW = NC * NS
PER_W = EE // NW
K = 80
ITERS = PER_W // K
WB = 80
NCH = NN // WB
CPS = -(-NCH // NS)
BE = 4000
GE = EE // BE


def _mesh():
    return plsc.VectorSubcoreMesh(core_axis_name="c", subcore_axis_name="s")


@functools.partial(jax.jit, static_argnames=("w",))
def _sc_gather(table, idx3, w):
    @functools.partial(
        pl.kernel,
        out_type=jax.ShapeDtypeStruct((EE, w), jnp.float32),
        mesh=_mesh(),
        scratch_types=[
            pltpu.VMEM((ITERS, K), jnp.int32),
            pltpu.VMEM((2, K, w), jnp.float32),
            pltpu.SemaphoreType.DMA,
            pltpu.SemaphoreType.DMA,
            pltpu.SemaphoreType.DMA,
            pltpu.SemaphoreType.DMA,
        ],
    )
    def k(table_hbm, idx_hbm, out_hbm, idx_v, rows_v, sem0, sem1, wsem0, wsem1):
        cid = lax.axis_index("c")
        sid = lax.axis_index("s")
        wid = sid * NC + cid
        base = wid * PER_W
        sems = (sem0, sem1)
        wsems = (wsem0, wsem1)

        pltpu.sync_copy(idx_hbm.at[wid], idx_v)
        pltpu.async_copy(table_hbm.at[idx_v.at[0]], rows_v.at[0], sems[0])

        def pair(pp, _):
            for b in range(2):
                i = pp * 2 + b

                @pl.when(i < ITERS)
                def _():
                    nb = 1 - b

                    @pl.when(i + 1 < ITERS)
                    def _():
                        @pl.when(i >= 1)
                        def _():
                            pltpu.make_async_copy(
                                rows_v.at[nb], out_hbm.at[pl.ds(base, K)], wsems[nb]).wait()

                        pltpu.async_copy(table_hbm.at[idx_v.at[i + 1]], rows_v.at[nb], sems[nb])

                    pltpu.make_async_copy(table_hbm.at[idx_v.at[i]], rows_v.at[b], sems[b]).wait()
                    pltpu.async_copy(rows_v.at[b], out_hbm.at[pl.ds(base + i * K, K)], wsems[b])

            return 0

        lax.fori_loop(0, (ITERS + 1) // 2, pair, 0)
        pltpu.make_async_copy(rows_v.at[0], out_hbm.at[pl.ds(base, K)], wsems[0]).wait()
        pltpu.make_async_copy(rows_v.at[1], out_hbm.at[pl.ds(base, K)], wsems[1]).wait()

    return k(table, idx3)


@functools.partial(jax.jit, static_argnames=("w",))
def _sc_scatter_add(rows, idx3, zchunk, w):
    @functools.partial(
        pl.kernel,
        out_type=jax.ShapeDtypeStruct((NC, NN, w), jnp.float32),
        mesh=_mesh(),
        scratch_types=[
            pltpu.VMEM((ITERS, K), jnp.int32),
            pltpu.VMEM((2, K, w), jnp.float32),
            pltpu.VMEM_SHARED((NN, w), jnp.float32),
            pltpu.SemaphoreType.DMA,
            pltpu.SemaphoreType.DMA,
        ],
    )
    def k(rows_hbm, idx_hbm, z_hbm, out_hbm, idx_v, rows_v, acc_sh, sem0, sem1):
        cid = lax.axis_index("c")
        sid = lax.axis_index("s")
        wid = sid * NC + cid
        base = wid * PER_W
        sems = (sem0, sem1)
        pltpu.sync_copy(idx_hbm.at[wid], idx_v)

        def zero(k, _):
            ch = sid * CPS + k

            @pl.when(ch < NCH)
            def _():
                pltpu.sync_copy(z_hbm, acc_sh.at[pl.ds(ch * WB, WB)])

            return 0

        lax.fori_loop(0, CPS, zero, 0)
        plsc.subcore_barrier()

        pltpu.async_copy(rows_hbm.at[pl.ds(base, K)], rows_v.at[0], sems[0])

        def pair(pp, _):
            for b in range(2):
                i = pp * 2 + b

                @pl.when(i < ITERS)
                def _():
                    nb = 1 - b

                    @pl.when(i + 1 < ITERS)
                    def _():
                        noff = base + (i + 1) * K
                        pltpu.async_copy(rows_hbm.at[pl.ds(noff, K)], rows_v.at[nb], sems[nb])

                    pltpu.make_async_copy(
                        rows_hbm.at[pl.ds(base + i * K, K)], rows_v.at[b], sems[b]).wait()
                    pltpu.sync_copy(rows_v.at[b], acc_sh.at[idx_v.at[i]], add=True)

            return 0

        lax.fori_loop(0, (ITERS + 1) // 2, pair, 0)
        plsc.subcore_barrier()

        def wout(k, _):
            ch = sid * CPS + k

            @pl.when(ch < NCH)
            def _():
                r0 = ch * WB
                pltpu.sync_copy(acc_sh.at[pl.ds(r0, WB)], out_hbm.at[cid, pl.ds(r0, WB)])

            return 0

        lax.fori_loop(0, CPS, wout, 0)

    return k(rows, idx3, zchunk)


@jax.jit
def _sc_rel(q128, src3, dst3):
    @functools.partial(
        pl.kernel,
        out_type=jax.ShapeDtypeStruct((EE, H), jnp.float32),
        mesh=_mesh(),
        scratch_types=[
            pltpu.VMEM((ITERS, K), jnp.int32),
            pltpu.VMEM((ITERS, K), jnp.int32),
            pltpu.VMEM((2, K, H), jnp.float32),
            pltpu.VMEM((2, K, H), jnp.float32),
            pltpu.SemaphoreType.DMA,
            pltpu.SemaphoreType.DMA,
            pltpu.SemaphoreType.DMA,
            pltpu.SemaphoreType.DMA,
            pltpu.SemaphoreType.DMA,
            pltpu.SemaphoreType.DMA,
        ],
    )
    def k(q_hbm, src_hbm, dst_hbm, out_hbm, ixs_v, ixd_v, qs_v, qd_v,
          ss0, ss1, sd0, sd1, ws0, ws1):
        cid = lax.axis_index("c")
        sid = lax.axis_index("s")
        wid = sid * NC + cid
        base = wid * PER_W
        ssems = (ss0, ss1)
        dsems = (sd0, sd1)
        wsems = (ws0, ws1)
        pltpu.sync_copy(src_hbm.at[wid], ixs_v)
        pltpu.sync_copy(dst_hbm.at[wid], ixd_v)

        def start(i, b):
            pltpu.async_copy(q_hbm.at[ixs_v.at[i]], qs_v.at[b], ssems[b])
            pltpu.async_copy(q_hbm.at[ixd_v.at[i]], qd_v.at[b], dsems[b])

        start(0, 0)

        def pair(pp, _):
            for b in range(2):
                i = pp * 2 + b

                @pl.when(i < ITERS)
                def _():
                    nb = 1 - b

                    @pl.when(i + 1 < ITERS)
                    def _():
                        @pl.when(i >= 1)
                        def _():
                            pltpu.make_async_copy(
                                qd_v.at[nb], out_hbm.at[pl.ds(base, K)], wsems[nb]).wait()

                        start(i + 1, nb)

                    pltpu.make_async_copy(q_hbm.at[ixs_v.at[i]], qs_v.at[b], ssems[b]).wait()
                    pltpu.make_async_copy(q_hbm.at[ixd_v.at[i]], qd_v.at[b], dsems[b]).wait()

                    def sub(r, _):
                        qd_v[b, r, pl.ds(0, 16)] = (qd_v[b, r, pl.ds(0, 16)]
                                                    - qs_v[b, r, pl.ds(0, 16)])
                        return 0

                    lax.fori_loop(0, K, sub, 0)
                    pltpu.async_copy(qd_v.at[b], out_hbm.at[pl.ds(base + i * K, K)], wsems[b])

            return 0

        lax.fori_loop(0, (ITERS + 1) // 2, pair, 0)
        pltpu.make_async_copy(qd_v.at[0], out_hbm.at[pl.ds(base, K)], wsems[0]).wait()
        pltpu.make_async_copy(qd_v.at[1], out_hbm.at[pl.ds(base, K)], wsems[1]).wait()

    return k(q128, src3, dst3)


@jax.jit
def _sc_scatter_pm(rows, src3, dst3, zchunk):
    @functools.partial(
        pl.kernel,
        out_type=jax.ShapeDtypeStruct((NC, NN, H), jnp.float32),
        mesh=_mesh(),
        scratch_types=[
            pltpu.VMEM((ITERS, K), jnp.int32),
            pltpu.VMEM((2, K), jnp.int32),
            pltpu.VMEM((2, K, H), jnp.float32),
            pltpu.VMEM_SHARED((NN, H), jnp.float32),
            pltpu.SemaphoreType.DMA,
            pltpu.SemaphoreType.DMA,
        ],
    )
    def k(rows_hbm, src_hbm, dst_hbm, z_hbm, out_hbm, ixs_v, ixd_v, rows_v,
          acc_sh, sem0, sem1):
        cid = lax.axis_index("c")
        sid = lax.axis_index("s")
        wid = sid * NC + cid
        base = wid * PER_W
        sems = (sem0, sem1)
        pltpu.sync_copy(src_hbm.at[wid], ixs_v)
        pltpu.sync_copy(dst_hbm.at[wid, 0], ixd_v.at[0])

        def zero(kk, _):
            ch = sid * CPS + kk

            @pl.when(ch < NCH)
            def _():
                pltpu.sync_copy(z_hbm, acc_sh.at[pl.ds(ch * WB, WB)])

            return 0

        lax.fori_loop(0, CPS, zero, 0)
        plsc.subcore_barrier()

        pltpu.async_copy(rows_hbm.at[pl.ds(base, K)], rows_v.at[0], sems[0])

        def pair(pp, _):
            for b in range(2):
                i = pp * 2 + b

                @pl.when(i < ITERS)
                def _():
                    nb = 1 - b

                    @pl.when(i + 1 < ITERS)
                    def _():
                        noff = base + (i + 1) * K
                        pltpu.async_copy(rows_hbm.at[pl.ds(noff, K)], rows_v.at[nb], sems[nb])
                        pltpu.sync_copy(dst_hbm.at[wid, i + 1], ixd_v.at[nb])

                    pltpu.make_async_copy(
                        rows_hbm.at[pl.ds(base + i * K, K)], rows_v.at[b], sems[b]).wait()
                    pltpu.sync_copy(rows_v.at[b], acc_sh.at[ixs_v.at[i]], add=True)

                    def neg(r, _):
                        rows_v[b, r, pl.ds(0, 16)] = -rows_v[b, r, pl.ds(0, 16)]
                        return 0

                    lax.fori_loop(0, K, neg, 0)
                    pltpu.sync_copy(rows_v.at[b], acc_sh.at[ixd_v.at[b]], add=True)

            return 0

        lax.fori_loop(0, (ITERS + 1) // 2, pair, 0)
        plsc.subcore_barrier()

        def wout(kk, _):
            ch = sid * CPS + kk

            @pl.when(ch < NCH)
            def _():
                r0 = ch * WB
                pltpu.sync_copy(acc_sh.at[pl.ds(r0, WB)], out_hbm.at[cid, pl.ds(r0, WB)])

            return 0

        lax.fori_loop(0, CPS, wout, 0)

    return k(rows, src3, dst3, zchunk)


def _sig(v):
    return jax.nn.sigmoid(v)


def _full(shape, dtype=jnp.float32):
    return jax.ShapeDtypeStruct(shape, dtype)


def _tc_prep(x, Wn, bn, A0):
    def body(x_r, wn_r, bn_r, a0_r, nf_r, nfa_r):
        nf = jnp.dot(x_r[...], wn_r[...], preferred_element_type=jnp.float32) + bn_r[...]
        nf_r[...] = nf
        nfa_r[...] = jnp.dot(nf, a0_r[...], preferred_element_type=jnp.float32)

    return pl.pallas_call(
        body, out_shape=[_full((NN, H)), _full((NN, H))])(x, Wn, bn.reshape(1, H), A0)


def _tc_dist(rel):
    def body(rel_r, d_r):
        rel_ = rel_r[...]
        ssq = jnp.sum(rel_ * rel_, axis=-1, keepdims=True)
        d_r[...] = jnp.sqrt(ssq + 1e-8)

    espec = pl.BlockSpec((BE, H), lambda i: (i, 0))
    return pl.pallas_call(
        body, grid=(GE,), in_specs=[espec],
        out_specs=pl.BlockSpec((BE, 1), lambda i: (i, 0)),
        out_shape=_full((EE, 1)))(rel)


def _tc_edge_fwd(ga, ea, dist, G, g0, c):
    def body(ga_r, ea_r, d_r, g_r, g0_r, c_r, msg_r, sigp_r):
        z = (ga_r[...] + jnp.dot(ea_r[...], g_r[...], preferred_element_type=jnp.float32)
             + g0_r[...] + d_r[...] * c_r[...])
        s = _sig(z)
        msg_r[...] = z * s
        sigp_r[...] = s * (1.0 + z * (1.0 - s))

    espec = pl.BlockSpec((BE, H), lambda i: (i, 0))
    return pl.pallas_call(
        body, grid=(GE,),
        in_specs=[espec,
                  pl.BlockSpec((BE, 16), lambda i: (i, 0)),
                  pl.BlockSpec((BE, 1), lambda i: (i, 0)),
                  pl.BlockSpec((16, H), lambda i: (0, 0)),
                  pl.BlockSpec((1, H), lambda i: (0, 0)),
                  pl.BlockSpec((1, H), lambda i: (0, 0))],
        out_specs=[espec, espec],
        out_shape=[_full((EE, H)), _full((EE, H))],
    )(ga, ea, dist, G, g0.reshape(1, H), c.reshape(1, H))


def _tc_node_fwd(nf, agg2, WuA, WuB, A_next):
    def body(nf_r, agg2_r, wua_r, wub_r, an_r, yln_r, istd_r, dsu_r, nfa_r):
        nf_ = nf_r[...]
        agg = agg2_r[0] + agg2_r[1]
        u = (jnp.dot(nf_, wua_r[...], preferred_element_type=jnp.float32)
             + jnp.dot(agg, wub_r[...], preferred_element_type=jnp.float32))
        s = _sig(u)
        upd = u * s
        dsu_r[...] = s * (1.0 + u * (1.0 - s))
        r = nf_ + upd
        m = jnp.mean(r, axis=-1, keepdims=True)
        cen = r - m
        var = jnp.mean(cen * cen, axis=-1, keepdims=True)
        istd = jax.lax.rsqrt(var + 1e-5)
        istd_r[...] = istd
        yln = cen * istd
        yln_r[...] = yln
        nfa_r[...] = jnp.dot(yln, an_r[...], preferred_element_type=jnp.float32)

    return pl.pallas_call(
        body,
        out_shape=[_full((NN, H)), _full((NN, 1)), _full((NN, H)), _full((NN, H))],
    )(nf, agg2, WuA, WuB, A_next)


def _tc_head(nf4, Wo, bo, Wp1, bp1, wp2row, Wp1T, WoT):
    def body(nf_r, wo_r, bo_r, wp1_r, bp1_r, wp2_r, wp1t_r, wot_r, dnf_r):
        out = jnp.dot(nf_r[...], wo_r[...], preferred_element_type=jnp.float32) + bo_r[...]
        o1 = jnp.dot(out, wp1_r[...], preferred_element_type=jnp.float32) + bp1_r[...]
        s = _sig(o1)
        do1 = wp2_r[...] * (s * (1.0 + o1 * (1.0 - s)))
        dout = jnp.dot(do1, wp1t_r[...], preferred_element_type=jnp.float32)
        dnf_r[...] = jnp.dot(dout, wot_r[...], preferred_element_type=jnp.float32)

    return pl.pallas_call(body, out_shape=_full((NN, H)))(
        nf4, Wo, bo.reshape(1, -1), Wp1, bp1.reshape(1, -1), wp2row, Wp1T, WoT)


def _tc_node_bwd(dnf, yln, istd, dsu, WuAT, WuBT):
    def body(dnf_r, yln_r, istd_r, dsu_r, wuat_r, wubt_r, dres_r, dagg_r):
        dnf_ = dnf_r[...]
        yln = yln_r[...]
        dr = istd_r[...] * (
            dnf_ - jnp.mean(dnf_, axis=-1, keepdims=True)
            - yln * jnp.mean(dnf_ * yln, axis=-1, keepdims=True))
        du = dr * dsu_r[...]
        dres_r[...] = dr + jnp.dot(du, wuat_r[...], preferred_element_type=jnp.float32)
        dagg_r[...] = jnp.dot(du, wubt_r[...], preferred_element_type=jnp.float32)

    return pl.pallas_call(
        body, out_shape=[_full((NN, H)), _full((NN, H))])(dnf, yln, istd, dsu, WuAT, WuBT)


def _tc_edge_bwd(gd, sigp, c, ddist_in):
    def body(gd_r, sigp_r, c_r, di_r, dz_r, do_r):
        dz = gd_r[...] * sigp_r[...]
        dz_r[...] = dz
        do_r[...] = di_r[...] + jnp.sum(dz * c_r[...], axis=-1, keepdims=True)

    espec = pl.BlockSpec((BE, H), lambda i: (i, 0))
    dspec = pl.BlockSpec((BE, 1), lambda i: (i, 0))
    return pl.pallas_call(
        body, grid=(GE,),
        in_specs=[espec, espec, pl.BlockSpec((1, H), lambda i: (0, 0)), dspec],
        out_specs=[espec, dspec],
        out_shape=[_full((EE, H)), _full((EE, 1))],
    )(gd, sigp, c.reshape(1, H), ddist_in)


def _tc_merge(dres, dnfa2, AT):
    def body(dres_r, dnfa2_r, at_r, dnf_r):
        dnfa = dnfa2_r[0] + dnfa2_r[1]
        dnf_r[...] = dres_r[...] + jnp.dot(dnfa, at_r[...], preferred_element_type=jnp.float32)

    return pl.pallas_call(body, out_shape=_full((NN, H)))(dres, dnfa2, AT)


def _tc_final_edge(rel, ddist):
    def body(rel_r, dd_r, drel_r):
        rel_ = rel_r[...]
        ssq = jnp.sum(rel_ * rel_, axis=-1, keepdims=True)
        dist = jnp.sqrt(ssq + 1e-8)
        drel_r[...] = (dd_r[...] / dist) * rel_

    espec = pl.BlockSpec((BE, H), lambda i: (i, 0))
    return pl.pallas_call(
        body, grid=(GE,),
        in_specs=[espec, pl.BlockSpec((BE, 1), lambda i: (i, 0))],
        out_specs=espec, out_shape=_full((EE, H)))(rel, ddist)


def _tc_finish(spm):
    def body(s_r, o_r):
        o_r[...] = s_r[0] + s_r[1]

    return pl.pallas_call(body, out_shape=_full((NN, H)))(spm)


def kernel(t, y, x, edge_index, edge_attr, Wn, bn, We, be, Wm, Wu, Wo, bo,
           Wp1, bp1, Wp2, bp2):
    src = edge_index[0].astype(jnp.int32)
    dst = edge_index[1].astype(jnp.int32)
    src3 = src.reshape(NW, ITERS, K)
    dst3 = dst.reshape(NW, ITERS, K)
    q = y[:, :3]
    p = y[:, 3:]
    q128 = jnp.pad(q, ((0, 0), (0, H - 3)))

    A = [Wm[l][:H] for l in range(LL)]
    G = [jnp.concatenate([Wm[l][H:H + 3], We @ Wm[l][H + 3:2 * H]], axis=0)
         for l in range(LL)]
    g0 = [be @ Wm[l][H + 3:2 * H] for l in range(LL)]
    c = [Wm[l][2 * H] for l in range(LL)]
    WuA = [Wu[l][:H] for l in range(LL)]
    WuB = [Wu[l][H:] for l in range(LL)]
    wp2row = jnp.broadcast_to(Wp2[:, 0], (1, Wp2.shape[0]))
    z128 = jnp.zeros((WB, H), jnp.float32)

    rel = _sc_rel(q128, src3, dst3)
    dist = _tc_dist(rel)

    nf, nfa = _tc_prep(x, Wn, bn, A[0])
    saves = []
    for l in range(LL):
        ga = _sc_gather(nfa, src3, H)
        msg, sigp = _tc_edge_fwd(ga, edge_attr, dist, G[l], g0[l], c[l])
        agg2 = _sc_scatter_add(msg, dst3, z128, H)
        A_next = A[l + 1] if l + 1 < LL else A[0]
        yln, istd, dsu, nfa = _tc_node_fwd(nf, agg2, WuA[l], WuB[l], A_next)
        saves.append((yln, istd, dsu, sigp))
        nf = yln

    dnf = _tc_head(nf, Wo, bo, Wp1, bp1, wp2row, Wp1.T, Wo.T)
    ddist = jnp.zeros((EE, 1), jnp.float32)
    for l in reversed(range(LL)):
        yln, istd, dsu, sigp = saves[l]
        dres, dagg = _tc_node_bwd(dnf, yln, istd, dsu, WuA[l].T, WuB[l].T)
        gd = _sc_gather(dagg, dst3, H)
        dz, ddist = _tc_edge_bwd(gd, sigp, c[l], ddist)
        if l > 0:
            dnfa2 = _sc_scatter_add(dz, src3, z128, H)
            dnf = _tc_merge(dres, dnfa2, A[l].T)

    drel = _tc_final_edge(rel, ddist)
    spm = _sc_scatter_pm(drel, src3, dst3, z128)
    gqneg = _tc_finish(spm)
    return jnp.concatenate([p, gqneg[:, :3]], axis=-1)


# speedup vs baseline: 3.5143x; 1.0367x over previous
import functools

import jax
import jax.numpy as jnp
from jax import lax
from jax.experimental import pallas as pl
from jax.experimental.pallas import tpu as pltpu
from jax.experimental.pallas import tpu_sc as plsc

NN = 10000
EE = 320000
H = 128
LL = 4

NC = 2
NS = 16
NW = NC * NS
PER_W = EE // NW
K = 80
ITERS = PER_W // K
WB = 80
NCH = NN // WB
CPS = -(-NCH // NS)
BE = 4000
GE = EE // BE


def _mesh():
    return plsc.VectorSubcoreMesh(core_axis_name="c", subcore_axis_name="s")


@functools.partial(jax.jit, static_argnames=("w", "dtype"))
def _sc_gather(table, idx3, w, dtype=jnp.float32):
    @functools.partial(
        pl.kernel,
        out_type=jax.ShapeDtypeStruct((EE, w), dtype),
        mesh=_mesh(),
        scratch_types=[
            pltpu.VMEM((ITERS, K), jnp.int32),
            pltpu.VMEM((2, K, w), dtype),
            pltpu.SemaphoreType.DMA,
            pltpu.SemaphoreType.DMA,
            pltpu.SemaphoreType.DMA,
            pltpu.SemaphoreType.DMA,
        ],
    )
    def k(table_hbm, idx_hbm, out_hbm, idx_v, rows_v, sem0, sem1, wsem0, wsem1):
        cid = lax.axis_index("c")
        sid = lax.axis_index("s")
        wid = sid * NC + cid
        base = wid * PER_W
        sems = (sem0, sem1)
        wsems = (wsem0, wsem1)

        pltpu.sync_copy(idx_hbm.at[wid], idx_v)
        pltpu.async_copy(table_hbm.at[idx_v.at[0]], rows_v.at[0], sems[0])

        def pair(pp, _):
            for b in range(2):
                i = pp * 2 + b

                @pl.when(i < ITERS)
                def _():
                    nb = 1 - b

                    @pl.when(i + 1 < ITERS)
                    def _():
                        @pl.when(i >= 1)
                        def _():
                            pltpu.make_async_copy(
                                rows_v.at[nb], out_hbm.at[pl.ds(base, K)], wsems[nb]).wait()

                        pltpu.async_copy(table_hbm.at[idx_v.at[i + 1]], rows_v.at[nb], sems[nb])

                    pltpu.make_async_copy(table_hbm.at[idx_v.at[i]], rows_v.at[b], sems[b]).wait()
                    pltpu.async_copy(rows_v.at[b], out_hbm.at[pl.ds(base + i * K, K)], wsems[b])

            return 0

        lax.fori_loop(0, (ITERS + 1) // 2, pair, 0)
        pltpu.make_async_copy(rows_v.at[0], out_hbm.at[pl.ds(base, K)], wsems[0]).wait()
        pltpu.make_async_copy(rows_v.at[1], out_hbm.at[pl.ds(base, K)], wsems[1]).wait()

    return k(table, idx3)


@functools.partial(jax.jit, static_argnames=("w",))
def _sc_scatter_add(rows, idx3, zchunk, w):
    @functools.partial(
        pl.kernel,
        out_type=jax.ShapeDtypeStruct((NC, NN, w), jnp.float32),
        mesh=_mesh(),
        scratch_types=[
            pltpu.VMEM((ITERS, K), jnp.int32),
            pltpu.VMEM((2, K, w), jnp.float32),
            pltpu.VMEM_SHARED((NN, w), jnp.float32),
            pltpu.SemaphoreType.DMA,
            pltpu.SemaphoreType.DMA,
        ],
    )
    def k(rows_hbm, idx_hbm, z_hbm, out_hbm, idx_v, rows_v, acc_sh, sem0, sem1):
        cid = lax.axis_index("c")
        sid = lax.axis_index("s")
        wid = sid * NC + cid
        base = wid * PER_W
        sems = (sem0, sem1)
        pltpu.sync_copy(idx_hbm.at[wid], idx_v)

        def zero(k, _):
            ch = sid * CPS + k

            @pl.when(ch < NCH)
            def _():
                pltpu.sync_copy(z_hbm, acc_sh.at[pl.ds(ch * WB, WB)])

            return 0

        lax.fori_loop(0, CPS, zero, 0)
        plsc.subcore_barrier()

        pltpu.async_copy(rows_hbm.at[pl.ds(base, K)], rows_v.at[0], sems[0])

        def pair(pp, _):
            for b in range(2):
                i = pp * 2 + b

                @pl.when(i < ITERS)
                def _():
                    nb = 1 - b

                    @pl.when(i + 1 < ITERS)
                    def _():
                        noff = base + (i + 1) * K
                        pltpu.async_copy(rows_hbm.at[pl.ds(noff, K)], rows_v.at[nb], sems[nb])

                    pltpu.make_async_copy(
                        rows_hbm.at[pl.ds(base + i * K, K)], rows_v.at[b], sems[b]).wait()
                    pltpu.sync_copy(rows_v.at[b], acc_sh.at[idx_v.at[i]], add=True)

            return 0

        lax.fori_loop(0, (ITERS + 1) // 2, pair, 0)
        plsc.subcore_barrier()

        def wout(k, _):
            ch = sid * CPS + k

            @pl.when(ch < NCH)
            def _():
                r0 = ch * WB
                pltpu.sync_copy(acc_sh.at[pl.ds(r0, WB)], out_hbm.at[cid, pl.ds(r0, WB)])

            return 0

        lax.fori_loop(0, CPS, wout, 0)

    return k(rows, idx3, zchunk)


@jax.jit
def _sc_rel(q128, src3, dst3):
    @functools.partial(
        pl.kernel,
        out_type=jax.ShapeDtypeStruct((EE, H), jnp.float32),
        mesh=_mesh(),
        scratch_types=[
            pltpu.VMEM((ITERS, K), jnp.int32),
            pltpu.VMEM((ITERS, K), jnp.int32),
            pltpu.VMEM((2, K, H), jnp.float32),
            pltpu.VMEM((2, K, H), jnp.float32),
            pltpu.SemaphoreType.DMA,
            pltpu.SemaphoreType.DMA,
            pltpu.SemaphoreType.DMA,
            pltpu.SemaphoreType.DMA,
            pltpu.SemaphoreType.DMA,
            pltpu.SemaphoreType.DMA,
        ],
    )
    def k(q_hbm, src_hbm, dst_hbm, out_hbm, ixs_v, ixd_v, qs_v, qd_v,
          ss0, ss1, sd0, sd1, ws0, ws1):
        cid = lax.axis_index("c")
        sid = lax.axis_index("s")
        wid = sid * NC + cid
        base = wid * PER_W
        ssems = (ss0, ss1)
        dsems = (sd0, sd1)
        wsems = (ws0, ws1)
        pltpu.sync_copy(src_hbm.at[wid], ixs_v)
        pltpu.sync_copy(dst_hbm.at[wid], ixd_v)

        def start(i, b):
            pltpu.async_copy(q_hbm.at[ixs_v.at[i]], qs_v.at[b], ssems[b])
            pltpu.async_copy(q_hbm.at[ixd_v.at[i]], qd_v.at[b], dsems[b])

        start(0, 0)

        def pair(pp, _):
            for b in range(2):
                i = pp * 2 + b

                @pl.when(i < ITERS)
                def _():
                    nb = 1 - b

                    @pl.when(i + 1 < ITERS)
                    def _():
                        @pl.when(i >= 1)
                        def _():
                            pltpu.make_async_copy(
                                qd_v.at[nb], out_hbm.at[pl.ds(base, K)], wsems[nb]).wait()

                        start(i + 1, nb)

                    pltpu.make_async_copy(q_hbm.at[ixs_v.at[i]], qs_v.at[b], ssems[b]).wait()
                    pltpu.make_async_copy(q_hbm.at[ixd_v.at[i]], qd_v.at[b], dsems[b]).wait()

                    def sub(r, _):
                        qd_v[b, r, pl.ds(0, 16)] = (qd_v[b, r, pl.ds(0, 16)]
                                                    - qs_v[b, r, pl.ds(0, 16)])
                        return 0

                    lax.fori_loop(0, K, sub, 0)
                    pltpu.async_copy(qd_v.at[b], out_hbm.at[pl.ds(base + i * K, K)], wsems[b])

            return 0

        lax.fori_loop(0, (ITERS + 1) // 2, pair, 0)
        pltpu.make_async_copy(qd_v.at[0], out_hbm.at[pl.ds(base, K)], wsems[0]).wait()
        pltpu.make_async_copy(qd_v.at[1], out_hbm.at[pl.ds(base, K)], wsems[1]).wait()

    return k(q128, src3, dst3)


@jax.jit
def _sc_scatter_pm(rows, src3, dst3, zchunk):
    @functools.partial(
        pl.kernel,
        out_type=jax.ShapeDtypeStruct((NC, NN, H), jnp.float32),
        mesh=_mesh(),
        scratch_types=[
            pltpu.VMEM((ITERS, K), jnp.int32),
            pltpu.VMEM((2, K), jnp.int32),
            pltpu.VMEM((2, K, H), jnp.float32),
            pltpu.VMEM_SHARED((NN, H), jnp.float32),
            pltpu.SemaphoreType.DMA,
            pltpu.SemaphoreType.DMA,
        ],
    )
    def k(rows_hbm, src_hbm, dst_hbm, z_hbm, out_hbm, ixs_v, ixd_v, rows_v,
          acc_sh, sem0, sem1):
        cid = lax.axis_index("c")
        sid = lax.axis_index("s")
        wid = sid * NC + cid
        base = wid * PER_W
        sems = (sem0, sem1)
        pltpu.sync_copy(src_hbm.at[wid], ixs_v)
        pltpu.sync_copy(dst_hbm.at[wid, 0], ixd_v.at[0])

        def zero(kk, _):
            ch = sid * CPS + kk

            @pl.when(ch < NCH)
            def _():
                pltpu.sync_copy(z_hbm, acc_sh.at[pl.ds(ch * WB, WB)])

            return 0

        lax.fori_loop(0, CPS, zero, 0)
        plsc.subcore_barrier()

        pltpu.async_copy(rows_hbm.at[pl.ds(base, K)], rows_v.at[0], sems[0])

        def pair(pp, _):
            for b in range(2):
                i = pp * 2 + b

                @pl.when(i < ITERS)
                def _():
                    nb = 1 - b

                    @pl.when(i + 1 < ITERS)
                    def _():
                        noff = base + (i + 1) * K
                        pltpu.async_copy(rows_hbm.at[pl.ds(noff, K)], rows_v.at[nb], sems[nb])
                        pltpu.sync_copy(dst_hbm.at[wid, i + 1], ixd_v.at[nb])

                    pltpu.make_async_copy(
                        rows_hbm.at[pl.ds(base + i * K, K)], rows_v.at[b], sems[b]).wait()
                    pltpu.sync_copy(rows_v.at[b], acc_sh.at[ixs_v.at[i]], add=True)

                    def neg(r, _):
                        rows_v[b, r, pl.ds(0, 16)] = -rows_v[b, r, pl.ds(0, 16)]
                        return 0

                    lax.fori_loop(0, K, neg, 0)
                    pltpu.sync_copy(rows_v.at[b], acc_sh.at[ixd_v.at[b]], add=True)

            return 0

        lax.fori_loop(0, (ITERS + 1) // 2, pair, 0)
        plsc.subcore_barrier()

        def wout(kk, _):
            ch = sid * CPS + kk

            @pl.when(ch < NCH)
            def _():
                r0 = ch * WB
                pltpu.sync_copy(acc_sh.at[pl.ds(r0, WB)], out_hbm.at[cid, pl.ds(r0, WB)])

            return 0

        lax.fori_loop(0, CPS, wout, 0)

    return k(rows, src3, dst3, zchunk)


def _sig(v):
    return jax.nn.sigmoid(v)


def _full(shape, dtype=jnp.float32):
    return jax.ShapeDtypeStruct(shape, dtype)


def _tc_prep(x, Wn, bn, A0):
    def body(x_r, wn_r, bn_r, a0_r, nf_r, nfa_r):
        nf = jnp.dot(x_r[...], wn_r[...], preferred_element_type=jnp.float32) + bn_r[...]
        nf_r[...] = nf
        nfa_r[...] = jnp.dot(nf, a0_r[...], preferred_element_type=jnp.float32)

    return pl.pallas_call(
        body, out_shape=[_full((NN, H)), _full((NN, H))])(x, Wn, bn.reshape(1, H), A0)


def _tc_dist(rel):
    def body(rel_r, d_r):
        rel_ = rel_r[...]
        ssq = jnp.sum(rel_ * rel_, axis=-1, keepdims=True)
        d_r[...] = jnp.sqrt(ssq + 1e-8)

    espec = pl.BlockSpec((BE, H), lambda i: (i, 0))
    return pl.pallas_call(
        body, grid=(GE,), in_specs=[espec],
        out_specs=pl.BlockSpec((BE, 1), lambda i: (i, 0)),
        out_shape=_full((EE, 1)))(rel)


def _tc_edge_fwd(ga, ea, dist, G, g0, c):
    def body(ga_r, ea_r, d_r, g_r, g0_r, c_r, msg_r, sigp_r):
        z = (ga_r[...].astype(jnp.float32)
             + jnp.dot(ea_r[...], g_r[...], preferred_element_type=jnp.float32)
             + g0_r[...] + d_r[...] * c_r[...])
        s = _sig(z)
        msg_r[...] = z * s
        sigp_r[...] = (s * (1.0 + z * (1.0 - s))).astype(jnp.bfloat16)

    espec = pl.BlockSpec((BE, H), lambda i: (i, 0))
    return pl.pallas_call(
        body, grid=(GE,),
        in_specs=[espec,
                  pl.BlockSpec((BE, 16), lambda i: (i, 0)),
                  pl.BlockSpec((BE, 1), lambda i: (i, 0)),
                  pl.BlockSpec((16, H), lambda i: (0, 0)),
                  pl.BlockSpec((1, H), lambda i: (0, 0)),
                  pl.BlockSpec((1, H), lambda i: (0, 0))],
        out_specs=[espec, espec],
        out_shape=[_full((EE, H)), _full((EE, H), jnp.bfloat16)],
    )(ga, ea, dist, G, g0.reshape(1, H), c.reshape(1, H))


def _tc_node_fwd(nf, agg2, WuA, WuB, A_next):
    def body(nf_r, agg2_r, wua_r, wub_r, an_r, yln_r, istd_r, dsu_r, nfa_r):
        nf_ = nf_r[...]
        agg = agg2_r[0] + agg2_r[1]
        u = (jnp.dot(nf_, wua_r[...], preferred_element_type=jnp.float32)
             + jnp.dot(agg, wub_r[...], preferred_element_type=jnp.float32))
        s = _sig(u)
        upd = u * s
        dsu_r[...] = s * (1.0 + u * (1.0 - s))
        r = nf_ + upd
        m = jnp.mean(r, axis=-1, keepdims=True)
        cen = r - m
        var = jnp.mean(cen * cen, axis=-1, keepdims=True)
        istd = jax.lax.rsqrt(var + 1e-5)
        istd_r[...] = istd
        yln = cen * istd
        yln_r[...] = yln
        nfa_r[...] = jnp.dot(yln, an_r[...], preferred_element_type=jnp.float32)

    return pl.pallas_call(
        body,
        out_shape=[_full((NN, H)), _full((NN, 1)), _full((NN, H)), _full((NN, H))],
    )(nf, agg2, WuA, WuB, A_next)


def _tc_head(nf4, Wo, bo, Wp1, bp1, wp2row, Wp1T, WoT):
    def body(nf_r, wo_r, bo_r, wp1_r, bp1_r, wp2_r, wp1t_r, wot_r, dnf_r):
        out = jnp.dot(nf_r[...], wo_r[...], preferred_element_type=jnp.float32) + bo_r[...]
        o1 = jnp.dot(out, wp1_r[...], preferred_element_type=jnp.float32) + bp1_r[...]
        s = _sig(o1)
        do1 = wp2_r[...] * (s * (1.0 + o1 * (1.0 - s)))
        dout = jnp.dot(do1, wp1t_r[...], preferred_element_type=jnp.float32)
        dnf_r[...] = jnp.dot(dout, wot_r[...], preferred_element_type=jnp.float32)

    return pl.pallas_call(body, out_shape=_full((NN, H)))(
        nf4, Wo, bo.reshape(1, -1), Wp1, bp1.reshape(1, -1), wp2row, Wp1T, WoT)


def _tc_node_bwd(dnf, yln, istd, dsu, WuAT, WuBT):
    def body(dnf_r, yln_r, istd_r, dsu_r, wuat_r, wubt_r, dres_r, dagg_r):
        dnf_ = dnf_r[...]
        yln = yln_r[...]
        dr = istd_r[...] * (
            dnf_ - jnp.mean(dnf_, axis=-1, keepdims=True)
            - yln * jnp.mean(dnf_ * yln, axis=-1, keepdims=True))
        du = dr * dsu_r[...]
        dres_r[...] = dr + jnp.dot(du, wuat_r[...], preferred_element_type=jnp.float32)
        dagg_r[...] = jnp.dot(du, wubt_r[...], preferred_element_type=jnp.float32)

    return pl.pallas_call(
        body, out_shape=[_full((NN, H)), _full((NN, H))])(dnf, yln, istd, dsu, WuAT, WuBT)


def _tc_edge_bwd(gd, sigp, c, ddist_in):
    def body(gd_r, sigp_r, c_r, di_r, dz_r, do_r):
        dz = gd_r[...].astype(jnp.float32) * sigp_r[...].astype(jnp.float32)
        dz_r[...] = dz
        do_r[...] = di_r[...] + jnp.sum(dz * c_r[...], axis=-1, keepdims=True)

    espec = pl.BlockSpec((BE, H), lambda i: (i, 0))
    dspec = pl.BlockSpec((BE, 1), lambda i: (i, 0))
    return pl.pallas_call(
        body, grid=(GE,),
        in_specs=[espec, espec, pl.BlockSpec((1, H), lambda i: (0, 0)), dspec],
        out_specs=[espec, dspec],
        out_shape=[_full((EE, H)), _full((EE, 1))],
    )(gd, sigp, c.reshape(1, H), ddist_in)


def _tc_merge(dres, dnfa2, AT):
    def body(dres_r, dnfa2_r, at_r, dnf_r):
        dnfa = dnfa2_r[0] + dnfa2_r[1]
        dnf_r[...] = dres_r[...] + jnp.dot(dnfa, at_r[...], preferred_element_type=jnp.float32)

    return pl.pallas_call(body, out_shape=_full((NN, H)))(dres, dnfa2, AT)


def _tc_final_edge(rel, ddist):
    def body(rel_r, dd_r, drel_r):
        rel_ = rel_r[...]
        ssq = jnp.sum(rel_ * rel_, axis=-1, keepdims=True)
        dist = jnp.sqrt(ssq + 1e-8)
        drel_r[...] = (dd_r[...] / dist) * rel_

    espec = pl.BlockSpec((BE, H), lambda i: (i, 0))
    return pl.pallas_call(
        body, grid=(GE,),
        in_specs=[espec, pl.BlockSpec((BE, 1), lambda i: (i, 0))],
        out_specs=espec, out_shape=_full((EE, H)))(rel, ddist)


def _tc_finish(spm):
    def body(s_r, o_r):
        o_r[...] = s_r[0] + s_r[1]

    return pl.pallas_call(body, out_shape=_full((NN, H)))(spm)


def kernel(t, y, x, edge_index, edge_attr, Wn, bn, We, be, Wm, Wu, Wo, bo,
           Wp1, bp1, Wp2, bp2):
    src = edge_index[0].astype(jnp.int32)
    dst = edge_index[1].astype(jnp.int32)
    src3 = src.reshape(NW, ITERS, K)
    dst3 = dst.reshape(NW, ITERS, K)
    q = y[:, :3]
    p = y[:, 3:]
    q128 = jnp.pad(q, ((0, 0), (0, H - 3)))

    A = [Wm[l][:H] for l in range(LL)]
    G = [jnp.concatenate([Wm[l][H:H + 3], We @ Wm[l][H + 3:2 * H]], axis=0)
         for l in range(LL)]
    g0 = [be @ Wm[l][H + 3:2 * H] for l in range(LL)]
    c = [Wm[l][2 * H] for l in range(LL)]
    WuA = [Wu[l][:H] for l in range(LL)]
    WuB = [Wu[l][H:] for l in range(LL)]
    wp2row = jnp.broadcast_to(Wp2[:, 0], (1, Wp2.shape[0]))
    z128 = jnp.zeros((WB, H), jnp.float32)

    rel = _sc_rel(q128, src3, dst3)
    dist = _tc_dist(rel)

    nf, nfa = _tc_prep(x, Wn, bn, A[0])
    saves = []
    for l in range(LL):
        ga = _sc_gather(nfa, src3, H)
        msg, sigp = _tc_edge_fwd(ga, edge_attr, dist, G[l], g0[l], c[l])
        agg2 = _sc_scatter_add(msg, dst3, z128, H)
        A_next = A[l + 1] if l + 1 < LL else A[0]
        yln, istd, dsu, nfa = _tc_node_fwd(nf, agg2, WuA[l], WuB[l], A_next)
        saves.append((yln, istd, dsu, sigp))
        nf = yln

    dnf = _tc_head(nf, Wo, bo, Wp1, bp1, wp2row, Wp1.T, Wo.T)
    ddist = jnp.zeros((EE, 1), jnp.float32)
    for l in reversed(range(LL)):
        yln, istd, dsu, sigp = saves[l]
        dres, dagg = _tc_node_bwd(dnf, yln, istd, dsu, WuA[l].T, WuB[l].T)
        gd = _sc_gather(dagg, dst3, H)
        dz, ddist = _tc_edge_bwd(gd, sigp, c[l], ddist)
        if l > 0:
            dnfa2 = _sc_scatter_add(dz, src3, z128, H)
            dnf = _tc_merge(dres, dnfa2, A[l].T)

    drel = _tc_final_edge(rel, ddist)
    spm = _sc_scatter_pm(drel, src3, dst3, z128)
    gqneg = _tc_finish(spm)
    return jnp.concatenate([p, gqneg[:, :3]], axis=-1)
